# Initial kernel scaffold; baseline (speedup 1.0000x reference)
import jax
import jax.numpy as jnp
from jax.experimental import pallas as pl


def kernel(x, edge_index, batch, W1, b1, W2, b2, Wf1, bf1, Wf2, bf2):
    raise NotImplementedError("write your pallas kernel here")


# speedup vs baseline: 11.8063x; 11.8063x over previous
import functools

import jax
import jax.numpy as jnp
from jax import lax
from jax.experimental import pallas as pl
from jax.experimental.pallas import tpu as pltpu
from jax.experimental.pallas import tpu_sc as plsc

N = 10000
E = 320000
G = 256
DF = 128
DH = 64
NP = 10240
NROWS = 10368
DUMMY = NP
NC, NS, NW = 2, 16, 32
CH = 128
NCH = 80
EPW = NCH * CH
EPAD = NW * EPW
SLAB = NROWS // NS
BR = 1024
GRID = NP // BR

@functools.lru_cache(maxsize=None)
def _mesh():
    return plsc.VectorSubcoreMesh(
        core_axis_name="c", subcore_axis_name="s",
        num_cores=NC, num_subcores=NS)


def _deg_body(dst_hbm, ones_hbm, zeros_hbm, out_hbm, didx, ones_v, acc, sem):
    cid = lax.axis_index("c")
    sid = lax.axis_index("s")
    w = sid * NC + cid
    pltpu.sync_copy(zeros_hbm.at[pl.ds(sid * SLAB, SLAB)],
                    acc.at[pl.ds(sid * SLAB, SLAB)])
    pltpu.sync_copy(ones_hbm, ones_v)
    pltpu.sync_copy(dst_hbm.at[w], didx)
    plsc.subcore_barrier()

    def body(j, carry):
        pltpu.sync_copy(ones_v, acc.at[didx.at[j]], add=True)
        return carry

    lax.fori_loop(0, NCH, body, 0)
    plsc.subcore_barrier()
    pltpu.sync_copy(acc.at[pl.ds(sid * SLAB, SLAB)],
                    out_hbm.at[cid, pl.ds(sid * SLAB, SLAB)])


@functools.lru_cache(maxsize=None)
def _deg_kernel():
    return functools.partial(
        pl.kernel,
        out_type=jax.ShapeDtypeStruct((NC, NROWS, 16), jnp.float32),
        mesh=_mesh(),
        compiler_params=pltpu.CompilerParams(use_tc_tiling_on_sc=False),
        scratch_types=[
            pltpu.VMEM((NCH, CH), jnp.int32),
            pltpu.VMEM((CH, 16), jnp.float32),
            pltpu.VMEM_SHARED((NROWS, 16), jnp.float32),
            pltpu.SemaphoreType.DMA,
        ])(_deg_body)


def _make_agg(D):
    def _agg_body(hs_hbm, src_hbm, dst_hbm, zeros_hbm, out_hbm,
                  sidx, didx, rows0, rows1, acc, sem_a, sem_b):
        cid = lax.axis_index("c")
        sid = lax.axis_index("s")
        w = sid * NC + cid
        pltpu.sync_copy(zeros_hbm.at[pl.ds(sid * SLAB, SLAB)],
                        acc.at[pl.ds(sid * SLAB, SLAB)])
        pltpu.sync_copy(src_hbm.at[w], sidx)
        pltpu.sync_copy(dst_hbm.at[w], didx)
        plsc.subcore_barrier()

        pltpu.async_copy(hs_hbm.at[sidx.at[0]], rows0, sem_a)

        def body(j, carry):
            a = 2 * j
            pltpu.async_copy(hs_hbm.at[sidx.at[a + 1]], rows1, sem_b)
            pltpu.make_async_copy(hs_hbm.at[sidx.at[a]], rows0, sem_a).wait()
            pltpu.sync_copy(rows0, acc.at[didx.at[a]], add=True)

            @pl.when(j + 1 < NCH // 2)
            def _():
                pltpu.async_copy(hs_hbm.at[sidx.at[a + 2]], rows0, sem_a)

            pltpu.make_async_copy(hs_hbm.at[sidx.at[a + 1]], rows1, sem_b).wait()
            pltpu.sync_copy(rows1, acc.at[didx.at[a + 1]], add=True)
            return carry

        lax.fori_loop(0, NCH // 2, body, 0)
        plsc.subcore_barrier()
        pltpu.sync_copy(acc.at[pl.ds(sid * SLAB, SLAB)],
                        out_hbm.at[cid, pl.ds(sid * SLAB, SLAB)])

    return functools.partial(
        pl.kernel,
        out_type=jax.ShapeDtypeStruct((NC, NROWS, D), jnp.float32),
        mesh=_mesh(),
        compiler_params=pltpu.CompilerParams(use_tc_tiling_on_sc=False),
        scratch_types=[
            pltpu.VMEM((NCH, CH), jnp.int32),
            pltpu.VMEM((NCH, CH), jnp.int32),
            pltpu.VMEM((CH, D), jnp.float32),
            pltpu.VMEM((CH, D), jnp.float32),
            pltpu.VMEM_SHARED((NROWS, D), jnp.float32),
            pltpu.SemaphoreType.DMA,
            pltpu.SemaphoreType.DMA,
        ])(_agg_body)


_agg_cached = functools.lru_cache(maxsize=None)(_make_agg)


def _agg2_body(hs_a, hs_b, src_hbm, dst_hbm, zeros_hbm, out_hbm,
               sidx, didx, rows0, rows1, acc, sem_a, sem_b):
    cid = lax.axis_index("c")
    sid = lax.axis_index("s")
    w = sid * NC + cid
    pltpu.sync_copy(src_hbm.at[w], sidx)
    pltpu.sync_copy(dst_hbm.at[w], didx)
    for t, tbl in enumerate((hs_a, hs_b)):
        pltpu.sync_copy(zeros_hbm.at[pl.ds(sid * SLAB, SLAB)],
                        acc.at[pl.ds(sid * SLAB, SLAB)])
        plsc.subcore_barrier()
        pltpu.async_copy(tbl.at[sidx.at[0]], rows0, sem_a)

        def body(j, carry):
            a = 2 * j
            pltpu.async_copy(tbl.at[sidx.at[a + 1]], rows1, sem_b)
            pltpu.make_async_copy(tbl.at[sidx.at[a]], rows0, sem_a).wait()
            pltpu.sync_copy(rows0, acc.at[didx.at[a]], add=True)

            @pl.when(j + 1 < NCH // 2)
            def _():
                pltpu.async_copy(tbl.at[sidx.at[a + 2]], rows0, sem_a)

            pltpu.make_async_copy(tbl.at[sidx.at[a + 1]], rows1, sem_b).wait()
            pltpu.sync_copy(rows1, acc.at[didx.at[a + 1]], add=True)
            return carry

        lax.fori_loop(0, NCH // 2, body, 0)
        plsc.subcore_barrier()
        pltpu.sync_copy(acc.at[pl.ds(sid * SLAB, SLAB)],
                        out_hbm.at[t, cid, pl.ds(sid * SLAB, SLAB)])


@functools.lru_cache(maxsize=None)
def _agg2_kernel():
    return functools.partial(
        pl.kernel,
        out_type=jax.ShapeDtypeStruct((2, NC, NROWS, DH), jnp.float32),
        mesh=_mesh(),
        compiler_params=pltpu.CompilerParams(use_tc_tiling_on_sc=False),
        scratch_types=[
            pltpu.VMEM((NCH, CH), jnp.int32),
            pltpu.VMEM((NCH, CH), jnp.int32),
            pltpu.VMEM((CH, DH), jnp.float32),
            pltpu.VMEM((CH, DH), jnp.float32),
            pltpu.VMEM_SHARED((NROWS, DH), jnp.float32),
            pltpu.SemaphoreType.DMA,
            pltpu.SemaphoreType.DMA,
        ])(_agg2_body)


def _tc1_body(x_ref, w_ref, d0_ref, d1_ref, hs_ref, dinv_ref):
    dinv = lax.rsqrt(d0_ref[...] + d1_ref[...] + 1.0)
    h = jnp.dot(x_ref[...], w_ref[...], preferred_element_type=jnp.float32)
    hs_ref[...] = h * dinv
    dinv_ref[...] = dinv


def _tc1(x_p, W1, deg0, deg1):
    return pl.pallas_call(
        _tc1_body,
        grid=(GRID,),
        in_specs=[
            pl.BlockSpec((BR, DF), lambda i: (i, 0)),
            pl.BlockSpec((DF, DH), lambda i: (0, 0)),
            pl.BlockSpec((BR, 1), lambda i: (i, 0)),
            pl.BlockSpec((BR, 1), lambda i: (i, 0)),
        ],
        out_specs=[
            pl.BlockSpec((BR, DH), lambda i: (i, 0)),
            pl.BlockSpec((BR, 1), lambda i: (i, 0)),
        ],
        out_shape=[
            jax.ShapeDtypeStruct((NP, DH), jnp.float32),
            jax.ShapeDtypeStruct((NP, 1), jnp.float32),
        ])(x_p, W1, deg0, deg1)


def _tc2_body(p0_ref, p1_ref, hs_ref, dinv_ref, b1_ref, w2_ref,
              oa_ref, ob_ref):
    dinv = dinv_ref[...]
    t = dinv * (p0_ref[...] + p1_ref[...] + hs_ref[...]) + b1_ref[0:1, :]
    t = jnp.maximum(t, 0.0)
    hs2 = dinv * jnp.dot(t, w2_ref[...], preferred_element_type=jnp.float32)
    oa_ref[...] = hs2[:, :DH]
    ob_ref[...] = hs2[:, DH:]


def _tc2(p0, p1, hs1, dinv, b1b, W2):
    return pl.pallas_call(
        _tc2_body,
        grid=(GRID,),
        in_specs=[
            pl.BlockSpec((BR, DH), lambda i: (i, 0)),
            pl.BlockSpec((BR, DH), lambda i: (i, 0)),
            pl.BlockSpec((BR, DH), lambda i: (i, 0)),
            pl.BlockSpec((BR, 1), lambda i: (i, 0)),
            pl.BlockSpec((8, DH), lambda i: (0, 0)),
            pl.BlockSpec((DH, DF), lambda i: (0, 0)),
        ],
        out_specs=[
            pl.BlockSpec((BR, DH), lambda i: (i, 0)),
            pl.BlockSpec((BR, DH), lambda i: (i, 0)),
        ],
        out_shape=[
            jax.ShapeDtypeStruct((NP, DH), jnp.float32),
            jax.ShapeDtypeStruct((NP, DH), jnp.float32),
        ])(p0, p1, hs1, dinv, b1b, W2)


def _tc3_body(p0a_ref, p1a_ref, p0b_ref, p1b_ref, hsa_ref, hsb_ref,
              dinv_ref, b2_ref, batch_ref,
              wf1_ref, bf1_ref, wf2_ref, bf2_ref, o_ref, gsum_ref, cnt_ref):
    i = pl.program_id(0)

    @pl.when(i == 0)
    def _():
        gsum_ref[...] = jnp.zeros_like(gsum_ref)
        cnt_ref[...] = jnp.zeros_like(cnt_ref)

    dinv = dinv_ref[...]
    h3a = dinv * (p0a_ref[...] + p1a_ref[...] + hsa_ref[...]) + b2_ref[0:1, :DH]
    h3b = dinv * (p0b_ref[...] + p1b_ref[...] + hsb_ref[...]) + b2_ref[0:1, DH:]
    h3 = jnp.maximum(jnp.concatenate([h3a, h3b], axis=1), 0.0)

    gidx = lax.broadcasted_iota(jnp.int32, (BR, G), 1)
    onehot = (batch_ref[...] == gidx).astype(jnp.float32)
    gsum_ref[...] += lax.dot_general(
        onehot, h3, (((0,), (0,)), ((), ())),
        preferred_element_type=jnp.float32)
    ones_blk = jnp.ones((BR, DF), jnp.float32)
    cnt_ref[...] += lax.dot_general(
        onehot, ones_blk, (((0,), (0,)), ((), ())),
        preferred_element_type=jnp.float32)

    @pl.when(i == GRID - 1)
    def _():
        g = gsum_ref[...] / jnp.maximum(cnt_ref[...], 1.0)
        z1 = jnp.dot(g, wf1_ref[...], preferred_element_type=jnp.float32)
        z1 = jnp.maximum(z1 + bf1_ref[0:1, :], 0.0)
        z2 = jnp.dot(z1, wf2_ref[...], preferred_element_type=jnp.float32)
        z2 = z2 + bf2_ref[0:1, :]
        o_ref[...] = 1.0 / (1.0 + jnp.exp(-z2))


def _tc3(p0a, p1a, p0b, p1b, hs2a, hs2b, dinv, b2b, batch_p,
         Wf1, bf1b, Wf2p, bf2b):
    return pl.pallas_call(
        _tc3_body,
        grid=(GRID,),
        in_specs=[
            pl.BlockSpec((BR, DH), lambda i: (i, 0)),
            pl.BlockSpec((BR, DH), lambda i: (i, 0)),
            pl.BlockSpec((BR, DH), lambda i: (i, 0)),
            pl.BlockSpec((BR, DH), lambda i: (i, 0)),
            pl.BlockSpec((BR, DH), lambda i: (i, 0)),
            pl.BlockSpec((BR, DH), lambda i: (i, 0)),
            pl.BlockSpec((BR, 1), lambda i: (i, 0)),
            pl.BlockSpec((8, DF), lambda i: (0, 0)),
            pl.BlockSpec((BR, 1), lambda i: (i, 0)),
            pl.BlockSpec((DF, DH), lambda i: (0, 0)),
            pl.BlockSpec((8, DH), lambda i: (0, 0)),
            pl.BlockSpec((DH, DF), lambda i: (0, 0)),
            pl.BlockSpec((8, DF), lambda i: (0, 0)),
        ],
        out_specs=pl.BlockSpec((G, DF), lambda i: (0, 0)),
        out_shape=jax.ShapeDtypeStruct((G, DF), jnp.float32),
        scratch_shapes=[
            pltpu.VMEM((G, DF), jnp.float32),
            pltpu.VMEM((G, DF), jnp.float32),
        ])(p0a, p1a, p0b, p1b, hs2a, hs2b, dinv, b2b, batch_p,
           Wf1, bf1b, Wf2p, bf2b)


def kernel(x, edge_index, batch, W1, b1, W2, b2, Wf1, bf1, Wf2, bf2):
    src = edge_index[0].astype(jnp.int32)
    dst = edge_index[1].astype(jnp.int32)
    src_r = jnp.concatenate(
        [src, jnp.zeros((EPAD - E,), jnp.int32)]).reshape(NW, NCH, CH)
    dst_r = jnp.concatenate(
        [dst, jnp.full((EPAD - E,), DUMMY, jnp.int32)]).reshape(NW, NCH, CH)
    x_p = jnp.concatenate([x, jnp.zeros((NP - N, DF), x.dtype)], axis=0)
    batch_p = jnp.concatenate(
        [batch.astype(jnp.int32), jnp.full((NP - N,), G, jnp.int32)]
    ).reshape(NP, 1)

    ones16 = jnp.ones((CH, 16), jnp.float32)
    zeros16 = jnp.zeros((NROWS, 16), jnp.float32)
    zeros64 = jnp.zeros((NROWS, DH), jnp.float32)

    b1b = jnp.broadcast_to(b1[None, :], (8, DH))
    b2b = jnp.broadcast_to(b2[None, :], (8, DF))
    bf1b = jnp.broadcast_to(bf1[None, :], (8, DH))
    Wf2p = jnp.pad(Wf2, ((0, 0), (0, DF - 1)))
    bf2b = jnp.broadcast_to(
        jnp.pad(bf2[None, :], ((0, 0), (0, DF - 1))), (8, DF))

    degp = _deg_kernel()(dst_r, ones16, zeros16)
    deg0 = degp[0, :NP, 0:1]
    deg1 = degp[1, :NP, 0:1]

    hs1, dinv = _tc1(x_p, W1, deg0, deg1)

    agg1 = _agg_cached(DH)(hs1, src_r, dst_r, zeros64)
    hs2a, hs2b = _tc2(agg1[0, :NP], agg1[1, :NP], hs1, dinv, b1b, W2)

    agg2 = _agg2_kernel()(hs2a, hs2b, src_r, dst_r, zeros64)
    out = _tc3(agg2[0, 0, :NP], agg2[0, 1, :NP],
               agg2[1, 0, :NP], agg2[1, 1, :NP],
               hs2a, hs2b, dinv, b2b, batch_p,
               Wf1, bf1b, Wf2p, bf2b)
    return out[:, :1]
